# Initial kernel scaffold; baseline (speedup 1.0000x reference)
import jax
import jax.numpy as jnp
from jax.experimental import pallas as pl


def kernel(x, edge_index, batch, cell_type_batch, W, b):
    raise NotImplementedError("write your pallas kernel here")


# speedup vs baseline: 13.0339x; 13.0339x over previous
import functools
import jax
import jax.numpy as jnp
from jax import lax
from jax.experimental import pallas as pl
from jax.experimental.pallas import tpu as pltpu
from jax.experimental.pallas import tpu_sc as plsc

NC = 2
NS = 16
LANES = 16


def _sc_mesh():
    return plsc.VectorSubcoreMesh(core_axis_name="c", subcore_axis_name="s",
                                  num_cores=NC, num_subcores=NS)


def _node_span(n_nodes):
    span = (((n_nodes + NS - 1) // NS) + LANES - 1) // LANES * LANES
    last = n_nodes - span * (NS - 1)
    return span, last


def _make_deg_kernel(n_nodes, n_edges):
    assert n_edges % 128 == 0
    n_chunks = n_edges // 128
    span, _ = _node_span(n_nodes)
    n_pad = span * NS

    @functools.partial(
        pl.kernel,
        out_type=jax.ShapeDtypeStruct((NC, n_pad), jnp.float32),
        mesh=_sc_mesh(),
        scratch_types=[
            pltpu.VMEM_SHARED((n_pad,), jnp.float32),
            pltpu.VMEM((span,), jnp.float32),
            pltpu.VMEM((128,), jnp.float32),
            pltpu.VMEM((128,), jnp.int32),
        ],
    )
    def deg_kernel(dst_hbm, degp_hbm, deg_sh, zbuf, ones_v, idx_v):
        c = lax.axis_index("c")
        s = lax.axis_index("s")
        for k in range(span // LANES):
            zbuf[pl.ds(k * LANES, LANES)] = jnp.zeros((LANES,), jnp.float32)
        for k in range(128 // LANES):
            ones_v[pl.ds(k * LANES, LANES)] = jnp.ones((LANES,), jnp.float32)
        pltpu.sync_copy(zbuf, deg_sh.at[pl.ds(span * s, span)])
        plsc.subcore_barrier()

        w = c * NS + s
        lo = w * n_chunks // (NC * NS)
        hi = (w + 1) * n_chunks // (NC * NS)

        def body(j, _):
            pltpu.sync_copy(dst_hbm.at[pl.ds(j * 128, 128)], idx_v)
            pltpu.sync_copy(ones_v, deg_sh.at[idx_v], add=True)
            return ()

        lax.fori_loop(lo, hi, body, ())
        plsc.subcore_barrier()

        pltpu.sync_copy(deg_sh.at[pl.ds(span * s, span)], zbuf)
        pltpu.sync_copy(zbuf, degp_hbm.at[c, pl.ds(span * s, span)])

    return deg_kernel


def _tc_scale_matmul(x, W, degp, blk):
    n, d_in = x.shape
    d_out = W.shape[1]
    dh = d_out // NC

    def body(x_ref, w_ref, degp_ref, o_ref):
        deg = degp_ref[0] + degp_ref[1] + 1.0
        dinv = jnp.where(deg > 0, lax.rsqrt(deg), 0.0)
        h = jnp.dot(x_ref[...], w_ref[...], preferred_element_type=jnp.float32)
        g = h * dinv
        for cc in range(NC):
            o_ref[cc] = g[:, cc * dh:(cc + 1) * dh]

    return pl.pallas_call(
        body,
        grid=(n // blk,),
        in_specs=[
            pl.BlockSpec((blk, d_in), lambda i: (i, 0)),
            pl.BlockSpec((d_in, d_out), lambda i: (0, 0)),
            pl.BlockSpec((NC, blk, 1), lambda i: (0, i, 0)),
        ],
        out_specs=pl.BlockSpec((NC, blk, dh), lambda i: (0, i, 0)),
        out_shape=jax.ShapeDtypeStruct((NC, n, dh), jnp.float32),
    )(x, W, degp)


def _make_agg_kernel(n_nodes, n_edges, dh):
    assert n_edges % 128 == 0
    n_chunks = n_edges // 128
    span, last = _node_span(n_nodes)
    i_ch = 128

    @functools.partial(
        pl.kernel,
        out_type=jax.ShapeDtypeStruct((NC, n_nodes, dh), jnp.float32),
        mesh=_sc_mesh(),
        scratch_types=[
            pltpu.VMEM_SHARED((n_nodes, dh), jnp.float32),
            pltpu.VMEM((128, dh), jnp.float32),
            pltpu.VMEM((128,), jnp.int32),
            pltpu.VMEM((128,), jnp.int32),
            pltpu.SemaphoreType.DMA,
        ],
    )
    def agg_kernel(g_hbm, src_hbm, dst_hbm, acc_hbm,
                   acc_sh, rows, idx_s, idx_d, sem):
        c = lax.axis_index("c")
        s = lax.axis_index("s")
        row0 = s * span

        def init_chunk(rbase, cnt):
            pltpu.sync_copy(g_hbm.at[pl.ds(c * n_nodes + rbase, cnt)],
                            rows.at[pl.ds(0, cnt)])
            pltpu.sync_copy(rows.at[pl.ds(0, cnt)],
                            acc_sh.at[pl.ds(rbase, cnt)])

        def out_chunk(rbase, cnt):
            pltpu.sync_copy(acc_sh.at[pl.ds(rbase, cnt)], rows.at[pl.ds(0, cnt)])
            pltpu.sync_copy(rows.at[pl.ds(0, cnt)],
                            acc_hbm.at[c, pl.ds(rbase, cnt)])

        def my_chunks(fn):
            @pl.when(s < NS - 1)
            def _():
                def body(k, _):
                    fn(row0 + k * i_ch, i_ch)
                    return ()
                lax.fori_loop(0, span // i_ch, body, ())
            @pl.when(s == NS - 1)
            def _():
                for k in range(0, last - (last % i_ch), i_ch):
                    fn(row0 + k, i_ch)
                if last % i_ch:
                    fn(row0 + last - (last % i_ch), last % i_ch)

        my_chunks(init_chunk)
        plsc.subcore_barrier()

        off = c * n_nodes
        lo = s * n_chunks // NS
        hi = (s + 1) * n_chunks // NS

        def body(j, _):
            pltpu.sync_copy(src_hbm.at[pl.ds(j * 128, 128)], idx_s)
            for k in range(128 // LANES):
                sl = pl.ds(k * LANES, LANES)
                idx_s[sl] = idx_s[sl] + off
            pltpu.async_copy(g_hbm.at[idx_s], rows, sem).wait()
            pltpu.sync_copy(dst_hbm.at[pl.ds(j * 128, 128)], idx_d)
            pltpu.sync_copy(rows, acc_sh.at[idx_d], add=True)
            return ()

        lax.fori_loop(lo, hi, body, ())
        plsc.subcore_barrier()

        my_chunks(out_chunk)

    return agg_kernel


def _tc_pool(acc2, degp, ctb2, b2, n_types, blk):
    _, n, dh = acc2.shape
    d = NC * dh
    nk = n // blk

    def body(a_ref, degp_ref, t_ref, b_ref, o_ref, acc, cnt):
        k = pl.program_id(0)
        deg = degp_ref[0] + degp_ref[1] + 1.0
        dinv = jnp.where(deg > 0, lax.rsqrt(deg), 0.0)
        a = jnp.concatenate([a_ref[0], a_ref[1]], axis=1)
        r = jnp.maximum(a * dinv + b_ref[...], 0.0)
        tids = lax.broadcasted_iota(jnp.int32, (blk, n_types), 1)
        m = (t_ref[...] == tids).astype(jnp.float32)
        part = lax.dot_general(m, r, (((0,), (0,)), ((), ())),
                               preferred_element_type=jnp.float32,
                               precision=lax.Precision.HIGHEST)
        cpart = lax.dot_general(m, jnp.ones((blk, 1), jnp.float32),
                                (((0,), (0,)), ((), ())),
                                preferred_element_type=jnp.float32,
                                precision=lax.Precision.HIGHEST)

        @pl.when(k == 0)
        def _():
            acc[...] = jnp.zeros_like(acc)
            cnt[...] = jnp.zeros_like(cnt)

        acc[...] += part
        cnt[...] += cpart

        @pl.when(k == nk - 1)
        def _():
            o_ref[...] = acc[...] / jnp.maximum(cnt[...], 1.0)

    return pl.pallas_call(
        body,
        grid=(nk,),
        in_specs=[
            pl.BlockSpec((NC, blk, dh), lambda i: (0, i, 0)),
            pl.BlockSpec((NC, blk, 1), lambda i: (0, i, 0)),
            pl.BlockSpec((blk, 1), lambda i: (i, 0)),
            pl.BlockSpec((1, d), lambda i: (0, 0)),
        ],
        out_specs=pl.BlockSpec((n_types, d), lambda i: (0, 0)),
        out_shape=jax.ShapeDtypeStruct((n_types, d), jnp.float32),
        scratch_shapes=[
            pltpu.VMEM((n_types, d), jnp.float32),
            pltpu.VMEM((n_types, 1), jnp.float32),
        ],
    )(acc2, degp, ctb2, b2)


def kernel(x, edge_index, batch, cell_type_batch, W, b):
    n, d_in = x.shape
    d_out = W.shape[1]
    dh = d_out // NC
    e = edge_index.shape[1]
    n_types = 100

    src = edge_index[0]
    dst = edge_index[1]

    span, _ = _node_span(n)
    n_pad = span * NS
    degp = _make_deg_kernel(n, e)(dst)
    degp3 = degp.reshape(NC, n_pad, 1)
    g2 = _tc_scale_matmul(x, W, degp3, blk=1000)
    g_flat = g2.reshape(NC * n, dh)
    acc2 = _make_agg_kernel(n, e, dh)(g_flat, src, dst)
    pooled = _tc_pool(acc2, degp3,
                      cell_type_batch.reshape(n, 1).astype(jnp.int32),
                      b.reshape(1, d_out), n_types, blk=1000)
    return pooled
